# baseline (speedup 1.0000x reference)
import functools

import jax
import jax.numpy as jnp
from jax.experimental import pallas as pl
from jax.experimental.pallas import tpu as pltpu

ALPHA = 0.2
BPB = 8
WIN = 128


def _gat_kernel(offsets_ref, cxt_ref, mask_ref, feat_ref, w_ref, a2_ref,
                out_ref):
    g = pl.program_id(0)
    nn = cxt_ref.shape[1]
    nrows = feat_ref.shape[0]

    wa = jnp.dot(w_ref[:], a2_ref[:].T, preferred_element_type=jnp.float32)

    feat_prog = feat_ref[pl.ds(g * BPB * nn, BPB * nn), :]
    st_all = jnp.dot(feat_prog, wa, preferred_element_type=jnp.float32)

    iota_w = jax.lax.broadcasted_iota(jnp.int32, (1, 1, WIN), 2)
    mixed_parts = []
    for k in range(BPB):
        b = g * BPB + k
        off = offsets_ref[b]
        base = jnp.minimum((off // 8) * 8, nrows - WIN)
        r = off - base

        feat_b = feat_prog[k * nn:(k + 1) * nn, :]
        win = feat_ref[pl.ds(base, WIN), :]

        st = st_all[k * nn:(k + 1) * nn, :]
        tw = jnp.dot(win, wa[:, 1:2], preferred_element_type=jnp.float32)

        cxt = cxt_ref[k] + r
        m = mask_ref[k]

        ohm = jnp.where(cxt[:, :, None] == iota_w, m[:, :, None], 0.0)
        tg = jnp.dot(ohm.reshape(nn * 5, WIN), tw,
                     preferred_element_type=jnp.float32).reshape(nn, 5)

        e = jnp.concatenate([st[:, 0:1] + st[:, 1:2], st[:, 0:1] + tg], axis=1)
        e = jnp.where(e >= 0, e, ALPHA * e)
        e = e - jnp.max(e, axis=1, keepdims=True)
        ex = jnp.exp(e)
        attn = ex / jnp.sum(ex, axis=1, keepdims=True)

        s_mat = jnp.sum(ohm * attn[:, 1:, None], axis=1)
        mixed_parts.append(
            attn[:, 0:1] * feat_b
            + jnp.dot(s_mat, win, preferred_element_type=jnp.float32))

    mixed = jnp.concatenate(mixed_parts, axis=0)
    h = jnp.dot(mixed, w_ref[:], preferred_element_type=jnp.float32)
    out_ref[...] = jnp.where(h >= 0, h, jnp.exp(jnp.minimum(h, 0.0)) - 1.0)


@jax.jit
def _run(feature, cxt_idx, offsets, maskf, W, a2d):
    bs, nper = cxt_idx.shape[0], cxt_idx.shape[1]
    out_f = W.shape[1]
    grid_spec = pltpu.PrefetchScalarGridSpec(
        num_scalar_prefetch=1,
        grid=(bs // BPB,),
        in_specs=[
            pl.BlockSpec((BPB, nper, 5), lambda g, *_: (g, 0, 0)),
            pl.BlockSpec((BPB, nper, 5), lambda g, *_: (g, 0, 0)),
            pl.BlockSpec(feature.shape, lambda g, *_: (0, 0)),
            pl.BlockSpec(W.shape, lambda g, *_: (0, 0)),
            pl.BlockSpec(a2d.shape, lambda g, *_: (0, 0)),
        ],
        out_specs=pl.BlockSpec((BPB * nper, out_f), lambda g, *_: (g, 0)),
    )
    return pl.pallas_call(
        _gat_kernel,
        grid_spec=grid_spec,
        out_shape=jax.ShapeDtypeStruct((bs * nper, out_f), jnp.float32),
    )(offsets, cxt_idx, maskf, feature, W, a2d)


def kernel(feature, cxt_idx, offset_idx, cxt_idx_mask, bs, n, W, a):
    out_f = W.shape[1]
    maskf = (cxt_idx_mask
             & (jnp.asarray(bs) > 0)
             & (jnp.asarray(n) > 0)).astype(jnp.float32)
    offsets = offset_idx.reshape(-1).astype(jnp.int32)
    a2d = a.reshape(2, out_f)
    return _run(feature, cxt_idx, offsets, maskf, W, a2d)


# speedup vs baseline: 1.2383x; 1.2383x over previous
import functools

import jax
import jax.numpy as jnp
from jax.experimental import pallas as pl
from jax.experimental.pallas import tpu as pltpu

ALPHA = 0.2
BPB = 8
WIN = 128


def _gat_kernel(offsets_ref, cxt_ref, mask_ref, feat_ref, w_ref, a2_ref,
                out_ref):
    g = pl.program_id(0)
    nn = cxt_ref.shape[1]
    nrows = feat_ref.shape[0]

    wa = jnp.dot(w_ref[:], a2_ref[:].T, preferred_element_type=jnp.float32)

    feat_prog = feat_ref[pl.ds(g * BPB * nn, BPB * nn), :]
    st_all = jnp.dot(feat_prog, wa, preferred_element_type=jnp.float32)

    iota_w = jax.lax.broadcasted_iota(jnp.int32, (1, 1, WIN), 2)
    mixed_parts = []
    for k in range(BPB):
        b = g * BPB + k
        off = offsets_ref[b]
        base = jnp.minimum((off // 8) * 8, nrows - WIN)
        r = off - base

        feat_b = feat_prog[k * nn:(k + 1) * nn, :]
        win = feat_ref[pl.ds(base, WIN), :]

        st = st_all[k * nn:(k + 1) * nn, :]
        tw = jnp.dot(win, wa[:, 1:2], preferred_element_type=jnp.float32)

        cxt = cxt_ref[k] + r
        m = mask_ref[k]

        ohm = jnp.where(cxt[:, :, None] == iota_w, m[:, :, None], 0.0)
        tg = jnp.sum(ohm * tw[:, 0][None, None, :], axis=2)

        e = jnp.concatenate([st[:, 0:1] + st[:, 1:2], st[:, 0:1] + tg], axis=1)
        e = jnp.where(e >= 0, e, ALPHA * e)
        e = e - jnp.max(e, axis=1, keepdims=True)
        ex = jnp.exp(e)
        attn = ex / jnp.sum(ex, axis=1, keepdims=True)

        s_mat = jnp.sum(ohm * attn[:, 1:, None], axis=1)
        mixed_parts.append(
            attn[:, 0:1] * feat_b
            + jnp.dot(s_mat, win, preferred_element_type=jnp.float32))

    mixed = jnp.concatenate(mixed_parts, axis=0)
    h = jnp.dot(mixed, w_ref[:], preferred_element_type=jnp.float32)
    out_ref[...] = jnp.where(h >= 0, h, jnp.exp(jnp.minimum(h, 0.0)) - 1.0)


@jax.jit
def _run(feature, cxt_idx, offsets, maskf, W, a2d):
    bs, nper = cxt_idx.shape[0], cxt_idx.shape[1]
    out_f = W.shape[1]
    grid_spec = pltpu.PrefetchScalarGridSpec(
        num_scalar_prefetch=1,
        grid=(bs // BPB,),
        in_specs=[
            pl.BlockSpec((BPB, nper, 5), lambda g, *_: (g, 0, 0)),
            pl.BlockSpec((BPB, nper, 5), lambda g, *_: (g, 0, 0)),
            pl.BlockSpec(feature.shape, lambda g, *_: (0, 0)),
            pl.BlockSpec(W.shape, lambda g, *_: (0, 0)),
            pl.BlockSpec(a2d.shape, lambda g, *_: (0, 0)),
        ],
        out_specs=pl.BlockSpec((BPB * nper, out_f), lambda g, *_: (g, 0)),
    )
    return pl.pallas_call(
        _gat_kernel,
        grid_spec=grid_spec,
        out_shape=jax.ShapeDtypeStruct((bs * nper, out_f), jnp.float32),
    )(offsets, cxt_idx, maskf, feature, W, a2d)


def kernel(feature, cxt_idx, offset_idx, cxt_idx_mask, bs, n, W, a):
    out_f = W.shape[1]
    maskf = (cxt_idx_mask
             & (jnp.asarray(bs) > 0)
             & (jnp.asarray(n) > 0)).astype(jnp.float32)
    offsets = offset_idx.reshape(-1).astype(jnp.int32)
    a2d = a.reshape(2, out_f)
    return _run(feature, cxt_idx, offsets, maskf, W, a2d)


# speedup vs baseline: 1.6979x; 1.3711x over previous
import functools

import jax
import jax.numpy as jnp
from jax.experimental import pallas as pl
from jax.experimental.pallas import tpu as pltpu

ALPHA = 0.2
BPB = 8
WIN = 128


def _gat_kernel(offsets_ref, cxt_ref, mask_ref, feat_ref, w_ref, a2_ref,
                out_ref):
    g = pl.program_id(0)
    nn = cxt_ref.shape[2]
    nrows = feat_ref.shape[0]

    wa = jnp.dot(w_ref[:], a2_ref[:].T, preferred_element_type=jnp.float32)

    feat_prog = feat_ref[pl.ds(g * BPB * nn, BPB * nn), :]
    st_t = jax.lax.dot_general(wa, feat_prog, (((0,), (1,)), ((), ())),
                               preferred_element_type=jnp.float32)

    iota_sub = jax.lax.broadcasted_iota(jnp.int32, (1, WIN, 1), 1)
    eye = (jax.lax.broadcasted_iota(jnp.int32, (nn, nn), 0)
           == jax.lax.broadcasted_iota(jnp.int32, (nn, nn), 1))
    mixed_parts = []
    for k in range(BPB):
        b = g * BPB + k
        off = offsets_ref[b]
        base = jnp.minimum((off // 8) * 8, nrows - WIN)
        r = off - base

        feat_b = feat_prog[k * nn:(k + 1) * nn, :]
        win = feat_ref[pl.ds(base, WIN), :]
        tw = jnp.dot(win, wa[:, 1:2], preferred_element_type=jnp.float32)

        cxt = cxt_ref[k] + r
        m = mask_ref[k]

        ohm = jnp.where(cxt[:, None, :] == iota_sub, m[:, None, :], 0.0)
        tg = jnp.sum(ohm * tw[None, :, :], axis=1)

        s_row = st_t[0:1, k * nn:(k + 1) * nn]
        t_row = st_t[1:2, k * nn:(k + 1) * nn]
        e = jnp.concatenate([s_row + t_row, s_row + tg], axis=0)
        e = jnp.where(e >= 0, e, ALPHA * e)
        e = e - jnp.max(e, axis=0, keepdims=True)
        ex = jnp.exp(e)
        attn = ex / jnp.sum(ex, axis=0, keepdims=True)

        s_mat = ohm[0] * attn[1:2, :]
        for j in range(1, 5):
            s_mat = s_mat + ohm[j] * attn[j + 1:j + 2, :]
        s_self = jnp.where(eye, attn[0:1, :], 0.0)
        smx = jnp.concatenate([s_mat, s_self], axis=0)
        wfx = jnp.concatenate([win, feat_b], axis=0)
        mixed_parts.append(
            jax.lax.dot_general(smx, wfx, (((0,), (0,)), ((), ())),
                                preferred_element_type=jnp.float32))

    mixed = jnp.concatenate(mixed_parts, axis=0)
    h = jnp.dot(mixed, w_ref[:], preferred_element_type=jnp.float32)
    out_ref[...] = jnp.where(h >= 0, h, jnp.exp(jnp.minimum(h, 0.0)) - 1.0)


@jax.jit
def _run(feature, cxt_t, offsets, mask_t, W, a2d):
    bs, nper = cxt_t.shape[0], cxt_t.shape[2]
    out_f = W.shape[1]
    grid_spec = pltpu.PrefetchScalarGridSpec(
        num_scalar_prefetch=1,
        grid=(bs // BPB,),
        in_specs=[
            pl.BlockSpec((BPB, 5, nper), lambda g, *_: (g, 0, 0)),
            pl.BlockSpec((BPB, 5, nper), lambda g, *_: (g, 0, 0)),
            pl.BlockSpec(feature.shape, lambda g, *_: (0, 0)),
            pl.BlockSpec(W.shape, lambda g, *_: (0, 0)),
            pl.BlockSpec(a2d.shape, lambda g, *_: (0, 0)),
        ],
        out_specs=pl.BlockSpec((BPB * nper, out_f), lambda g, *_: (g, 0)),
    )
    return pl.pallas_call(
        _gat_kernel,
        grid_spec=grid_spec,
        out_shape=jax.ShapeDtypeStruct((bs * nper, out_f), jnp.float32),
    )(offsets, cxt_t, mask_t, feature, W, a2d)


def kernel(feature, cxt_idx, offset_idx, cxt_idx_mask, bs, n, W, a):
    out_f = W.shape[1]
    maskf = (cxt_idx_mask
             & (jnp.asarray(bs) > 0)
             & (jnp.asarray(n) > 0)).astype(jnp.float32)
    offsets = offset_idx.reshape(-1).astype(jnp.int32)
    a2d = a.reshape(2, out_f)
    return _run(feature, cxt_idx.transpose(0, 2, 1), offsets,
                maskf.transpose(0, 2, 1), W, a2d)


# speedup vs baseline: 2.2426x; 1.3208x over previous
import functools

import jax
import jax.numpy as jnp
from jax.experimental import pallas as pl
from jax.experimental.pallas import tpu as pltpu

ALPHA = 0.2
BPB = 8
WIN = 128


def _gat_kernel(offsets_ref, cxt_ref, mask_ref, feat_ref, w_ref, a2_ref,
                out_ref, wa_ref):
    g = pl.program_id(0)
    nn = cxt_ref.shape[2]
    nrows = feat_ref.shape[0]

    @pl.when(g == 0)
    def _():
        wa_ref[...] = jnp.dot(w_ref[:], a2_ref[:].T,
                              preferred_element_type=jnp.float32)

    wa = wa_ref[...]

    feat_prog = feat_ref[pl.ds(g * BPB * nn, BPB * nn), :]
    st_t = jax.lax.dot_general(wa, feat_prog, (((0,), (1,)), ((), ())),
                               preferred_element_type=jnp.float32)

    iota_sub = jax.lax.broadcasted_iota(jnp.int32, (1, WIN, 1), 1)
    eye = (jax.lax.broadcasted_iota(jnp.int32, (nn, nn), 0)
           == jax.lax.broadcasted_iota(jnp.int32, (nn, nn), 1))
    mixed_parts = []
    for k in range(BPB):
        b = g * BPB + k
        off = offsets_ref[b]
        base = jnp.minimum((off // 8) * 8, nrows - WIN)
        r = off - base

        feat_b = feat_prog[k * nn:(k + 1) * nn, :]
        win = feat_ref[pl.ds(base, WIN), :]
        tw = jnp.dot(win, wa[:, 1:2], preferred_element_type=jnp.float32)

        cxt = cxt_ref[k] + r
        m = mask_ref[k]

        ohm = jnp.where(cxt[:, None, :] == iota_sub, m[:, None, :], 0.0)
        tg = jnp.sum(ohm * tw[None, :, :], axis=1)

        s_row = st_t[0:1, k * nn:(k + 1) * nn]
        t_row = st_t[1:2, k * nn:(k + 1) * nn]
        e = jnp.concatenate([s_row + t_row, s_row + tg], axis=0)
        e = jnp.where(e >= 0, e, ALPHA * e)
        e = e - jnp.max(e, axis=0, keepdims=True)
        ex = jnp.exp(e)
        attn = ex / jnp.sum(ex, axis=0, keepdims=True)

        s_mat = ohm[0] * attn[1:2, :]
        for j in range(1, 5):
            s_mat = s_mat + ohm[j] * attn[j + 1:j + 2, :]
        s_self = jnp.where(eye, attn[0:1, :], 0.0)
        mixed_parts.append(
            jax.lax.dot_general(s_mat, win, (((0,), (0,)), ((), ())),
                                preferred_element_type=jnp.float32)
            + jax.lax.dot_general(s_self, feat_b, (((0,), (0,)), ((), ())),
                                  preferred_element_type=jnp.float32))

    mixed = jnp.concatenate(mixed_parts, axis=0)
    h = jnp.dot(mixed, w_ref[:], preferred_element_type=jnp.float32)
    out_ref[...] = jnp.where(h >= 0, h, jnp.exp(jnp.minimum(h, 0.0)) - 1.0)


@jax.jit
def _run(feature, cxt_t, offsets, mask_t, W, a2d):
    bs, nper = cxt_t.shape[0], cxt_t.shape[2]
    out_f = W.shape[1]
    grid_spec = pltpu.PrefetchScalarGridSpec(
        num_scalar_prefetch=1,
        grid=(bs // BPB,),
        in_specs=[
            pl.BlockSpec((BPB, 5, nper), lambda g, *_: (g, 0, 0)),
            pl.BlockSpec((BPB, 5, nper), lambda g, *_: (g, 0, 0)),
            pl.BlockSpec(feature.shape, lambda g, *_: (0, 0)),
            pl.BlockSpec(W.shape, lambda g, *_: (0, 0)),
            pl.BlockSpec(a2d.shape, lambda g, *_: (0, 0)),
        ],
        out_specs=pl.BlockSpec((BPB * nper, out_f), lambda g, *_: (g, 0)),
        scratch_shapes=[pltpu.VMEM((feature.shape[1], 2), jnp.float32)],
    )
    return pl.pallas_call(
        _gat_kernel,
        grid_spec=grid_spec,
        out_shape=jax.ShapeDtypeStruct((bs * nper, out_f), jnp.float32),
    )(offsets, cxt_t, mask_t, feature, W, a2d)


def kernel(feature, cxt_idx, offset_idx, cxt_idx_mask, bs, n, W, a):
    out_f = W.shape[1]
    maskf = (cxt_idx_mask
             & (jnp.asarray(bs) > 0)
             & (jnp.asarray(n) > 0)).astype(jnp.float32)
    offsets = offset_idx.reshape(-1).astype(jnp.int32)
    a2d = a.reshape(2, out_f)
    return _run(feature, cxt_idx.transpose(0, 2, 1), offsets,
                maskf.transpose(0, 2, 1), W, a2d)


# speedup vs baseline: 2.4811x; 1.1063x over previous
import functools

import jax
import jax.numpy as jnp
from jax.experimental import pallas as pl
from jax.experimental.pallas import tpu as pltpu

ALPHA = 0.2
BPB = 8
WIN = 72


def _gat_kernel(offsets_ref, cxt_ref, mask_ref, feat_ref, w_ref, a2_ref,
                out_ref, wa_ref):
    g = pl.program_id(0)
    nn = cxt_ref.shape[2]
    nrows = feat_ref.shape[0]

    @pl.when(g == 0)
    def _():
        wa_ref[...] = jnp.dot(w_ref[:], a2_ref[:].T,
                              preferred_element_type=jnp.float32)

    wa = wa_ref[...]

    feat_prog = feat_ref[pl.ds(g * BPB * nn, BPB * nn), :]
    st_t = jax.lax.dot_general(wa, feat_prog, (((0,), (1,)), ((), ())),
                               preferred_element_type=jnp.float32)

    iota_sub = jax.lax.broadcasted_iota(jnp.int32, (1, WIN, 1), 1)
    eye = (jax.lax.broadcasted_iota(jnp.int32, (nn, nn), 0)
           == jax.lax.broadcasted_iota(jnp.int32, (nn, nn), 1))
    mixed_parts = []
    for k in range(BPB):
        b = g * BPB + k
        off = offsets_ref[b]
        base = jnp.minimum((off // 8) * 8, nrows - WIN)
        r = off - base

        feat_b = feat_prog[k * nn:(k + 1) * nn, :]
        win = feat_ref[pl.ds(base, WIN), :]
        tw = jnp.dot(win, wa[:, 1:2], preferred_element_type=jnp.float32)

        cxt = cxt_ref[k] + r
        m = mask_ref[k]

        ohm = jnp.where(cxt[:, None, :] == iota_sub, m[:, None, :], 0.0)
        tg = jnp.sum(ohm * tw[None, :, :], axis=1)

        s_row = st_t[0:1, k * nn:(k + 1) * nn]
        t_row = st_t[1:2, k * nn:(k + 1) * nn]
        e = jnp.concatenate([s_row + t_row, s_row + tg], axis=0)
        e = jnp.where(e >= 0, e, ALPHA * e)
        e = e - jnp.max(e, axis=0, keepdims=True)
        ex = jnp.exp(e)
        attn = ex / jnp.sum(ex, axis=0, keepdims=True)

        s_mat = ohm[0] * attn[1:2, :]
        for j in range(1, 5):
            s_mat = s_mat + ohm[j] * attn[j + 1:j + 2, :]
        s_self = jnp.where(eye, attn[0:1, :], 0.0)
        mixed_parts.append(
            jax.lax.dot_general(s_mat, win, (((0,), (0,)), ((), ())),
                                preferred_element_type=jnp.float32)
            + jax.lax.dot_general(s_self, feat_b, (((0,), (0,)), ((), ())),
                                  preferred_element_type=jnp.float32))

    mixed = jnp.concatenate(mixed_parts, axis=0)
    h = jnp.dot(mixed, w_ref[:], preferred_element_type=jnp.float32)
    out_ref[...] = jnp.where(h >= 0, h, jnp.exp(jnp.minimum(h, 0.0)) - 1.0)


@jax.jit
def _run(feature, cxt_t, offsets, mask_t, W, a2d):
    bs, nper = cxt_t.shape[0], cxt_t.shape[2]
    out_f = W.shape[1]
    grid_spec = pltpu.PrefetchScalarGridSpec(
        num_scalar_prefetch=1,
        grid=(bs // BPB,),
        in_specs=[
            pl.BlockSpec((BPB, 5, nper), lambda g, *_: (g, 0, 0)),
            pl.BlockSpec((BPB, 5, nper), lambda g, *_: (g, 0, 0)),
            pl.BlockSpec(feature.shape, lambda g, *_: (0, 0)),
            pl.BlockSpec(W.shape, lambda g, *_: (0, 0)),
            pl.BlockSpec(a2d.shape, lambda g, *_: (0, 0)),
        ],
        out_specs=pl.BlockSpec((BPB * nper, out_f), lambda g, *_: (g, 0)),
        scratch_shapes=[pltpu.VMEM((feature.shape[1], 2), jnp.float32)],
    )
    return pl.pallas_call(
        _gat_kernel,
        grid_spec=grid_spec,
        out_shape=jax.ShapeDtypeStruct((bs * nper, out_f), jnp.float32),
    )(offsets, cxt_t, mask_t, feature, W, a2d)


def kernel(feature, cxt_idx, offset_idx, cxt_idx_mask, bs, n, W, a):
    out_f = W.shape[1]
    maskf = (cxt_idx_mask
             & (jnp.asarray(bs) > 0)
             & (jnp.asarray(n) > 0)).astype(jnp.float32)
    offsets = offset_idx.reshape(-1).astype(jnp.int32)
    a2d = a.reshape(2, out_f)
    return _run(feature, cxt_idx.transpose(0, 2, 1), offsets,
                maskf.transpose(0, 2, 1), W, a2d)
